# baseline (speedup 1.0000x reference)
import jax
import jax.numpy as jnp
from jax.experimental import pallas as pl
from jax.experimental.pallas import tpu as pltpu

B = 16384
INPUT_DIM = 512
EMB_DIM = 128
N_CLUSTERS = 64

BM = 4096
HALF = BM // 2
NB = B // BM


def _fused_body(x1_ref, x2_ref, w_ref, c_ref, out_ref):
    w = w_ref[...]
    c = c_ref[...]
    c2 = jnp.sum(c * c, axis=1, keepdims=True)
    outs = []
    for xr in (x1_ref, x2_ref):
        xb = xr[...]
        xeT = jax.lax.dot_general(
            w, xb, (((1,), (1,)), ((), ())), preferred_element_type=jnp.float32
        )
        x2v = jnp.sum(xeT * xeT, axis=0, keepdims=True)
        xcT = jax.lax.dot_general(
            c, xeT, (((1,), (0,)), ((), ())), preferred_element_type=jnp.float32
        )
        m = jnp.min(c2 - 2.0 * xcT, axis=0, keepdims=True)
        outs.append(jnp.sqrt(jnp.maximum(m + x2v, 0.0)))
    out_ref[...] = jnp.concatenate(outs, axis=1)[None]


@jax.jit
def kernel(x, pca_components, centroids):
    out = pl.pallas_call(
        _fused_body,
        grid=(NB,),
        in_specs=[
            pl.BlockSpec((HALF, INPUT_DIM), lambda i: (2 * i, 0)),
            pl.BlockSpec((HALF, INPUT_DIM), lambda i: (2 * i + 1, 0)),
            pl.BlockSpec((EMB_DIM, INPUT_DIM), lambda i: (0, 0)),
            pl.BlockSpec((N_CLUSTERS, EMB_DIM), lambda i: (0, 0)),
        ],
        out_specs=pl.BlockSpec((1, 1, BM), lambda i: (i, 0, 0)),
        out_shape=jax.ShapeDtypeStruct((NB, 1, BM), jnp.float32),
        compiler_params=pltpu.CompilerParams(
            dimension_semantics=("parallel",),
        ),
    )(x, x, pca_components, centroids)
    return out.reshape(B)
